# baseline (speedup 1.0000x reference)
import jax
import jax.numpy as jnp
from jax import lax
from jax.experimental import pallas as pl
from jax.experimental.pallas import tpu as pltpu
from jax.experimental.pallas import tpu_sc as plsc

N_NODES = 100000
D = 128
NUM_SEGMENTS = 1024
EPS = 1e-5

NC = 2
NS = 16
NW = NC * NS
R = 160
NCHUNK = N_NODES // R
SCW = 80
NSC = R // SCW
CNTW = 16
NB = 4


def _sc_partial_sums(x, batch3, ones_hbm, z_sums, z_cnt):
    mesh = plsc.VectorSubcoreMesh(core_axis_name="c", subcore_axis_name="s")

    @pl.kernel(
        out_type=[
            jax.ShapeDtypeStruct((NC, NUM_SEGMENTS, D), jnp.float32),
            jax.ShapeDtypeStruct((NC, NUM_SEGMENTS, CNTW), jnp.float32),
        ],
        mesh=mesh,
        scratch_types=[
            pltpu.VMEM((NB, R, D), jnp.float32),
            pltpu.VMEM((NB, NSC, SCW), jnp.int32),
            pltpu.VMEM((SCW, CNTW), jnp.float32),
            pltpu.VMEM_SHARED((NUM_SEGMENTS, D), jnp.float32),
            pltpu.VMEM_SHARED((NUM_SEGMENTS, CNTW), jnp.float32),
            pltpu.SemaphoreType.DMA,
            pltpu.SemaphoreType.DMA,
            pltpu.SemaphoreType.DMA,
            pltpu.SemaphoreType.DMA,
            pltpu.SemaphoreType.DMA,
        ],
    )
    def k(x_hbm, b_hbm, ones_h, zs_h, zc_h, sums_out, cnts_out,
          xbuf, idxbuf, onesbuf, sums_sh, cnts_sh,
          sem0, sem1, sem2, sem3, sem_sc):
        cid = lax.axis_index("c")
        sid = lax.axis_index("s")
        wid = sid * NC + cid
        sems = (sem0, sem1, sem2, sem3)

        @pl.when(sid == 0)
        def _():
            pltpu.sync_copy(zs_h, sums_sh)
            pltpu.sync_copy(zc_h, cnts_sh)

        pltpu.sync_copy(ones_h, onesbuf)
        plsc.subcore_barrier()

        lo = (wid * NCHUNK) // NW
        hi = ((wid + 1) * NCHUNK) // NW

        def start_in(chunk, b):
            pltpu.async_copy(x_hbm.at[pl.ds(chunk * R, R)], xbuf.at[b], sems[b])
            pltpu.async_copy(b_hbm.at[chunk], idxbuf.at[b], sems[b])

        def wait_in(chunk, b):
            pltpu.make_async_copy(x_hbm.at[pl.ds(chunk * R, R)], xbuf.at[b],
                                  sems[b]).wait()
            pltpu.make_async_copy(b_hbm.at[chunk], idxbuf.at[b],
                                  sems[b]).wait()

        def fire_scatters(b):
            xv = xbuf.at[b]
            iv = idxbuf.at[b]
            for j in range(NSC):
                pltpu.async_copy(xv.at[pl.ds(j * SCW, SCW)],
                                 sums_sh.at[iv.at[j]], sem_sc, add=True)
                pltpu.async_copy(onesbuf, cnts_sh.at[iv.at[j]], sem_sc,
                                 add=True)

        def drain_scatters(b):
            xv = xbuf.at[b]
            iv = idxbuf.at[b]
            for j in range(NSC):
                pltpu.make_async_copy(xv.at[pl.ds(j * SCW, SCW)],
                                      sums_sh.at[iv.at[j]], sem_sc).wait()
                pltpu.make_async_copy(onesbuf, cnts_sh.at[iv.at[j]],
                                      sem_sc).wait()

        start_in(lo, 0)
        start_in(lo + 1, 1)

        n_outer = (hi - lo + NB - 1) // NB

        def body(kk, carry):
            i = lo + NB * kk
            for b in range(NB):
                chunk = i + b

                @pl.when(chunk < hi)
                def _():
                    bn = (b + 2) % NB

                    @pl.when(chunk - 2 >= lo)
                    def _():
                        drain_scatters(bn)

                    @pl.when(chunk + 2 < hi)
                    def _():
                        start_in(chunk + 2, bn)

                    wait_in(chunk, b)
                    fire_scatters(b)
            return carry

        lax.fori_loop(0, n_outer, body, 0)

        for _t in range(2):
            drain_scatters(0)
        plsc.subcore_barrier()

        @pl.when(sid == 0)
        def _():
            pltpu.sync_copy(sums_sh, sums_out.at[cid])
            pltpu.sync_copy(cnts_sh, cnts_out.at[cid])

    return k(x, batch3, ones_hbm, z_sums, z_cnt)


def _finish(sums_ref, cnts_ref, w_ref, b_ref, o_ref):
    s = sums_ref[0] + sums_ref[1]
    c = cnts_ref[0, :, 0:1] + cnts_ref[1, :, 0:1]
    h = s / jnp.maximum(c, 1.0)
    mu = jnp.mean(h, axis=1, keepdims=True)
    var = jnp.mean((h - mu) ** 2, axis=1, keepdims=True)
    o_ref[...] = (h - mu) * lax.rsqrt(var + EPS) * w_ref[0] + b_ref[0]


def kernel(x, batch, ln_weight, ln_bias):
    batch3 = batch.astype(jnp.int32).reshape(NCHUNK, NSC, SCW)
    ones_hbm = jnp.ones((SCW, CNTW), dtype=jnp.float32)
    z_sums = jnp.zeros((NUM_SEGMENTS, D), dtype=jnp.float32)
    z_cnt = jnp.zeros((NUM_SEGMENTS, CNTW), dtype=jnp.float32)

    sums_p, cnts_p = _sc_partial_sums(x, batch3, ones_hbm, z_sums, z_cnt)

    return pl.pallas_call(
        _finish,
        out_shape=jax.ShapeDtypeStruct((NUM_SEGMENTS, D), jnp.float32),
    )(sums_p, cnts_p, ln_weight.reshape(1, D), ln_bias.reshape(1, D))


# speedup vs baseline: 1.0159x; 1.0159x over previous
import jax
import jax.numpy as jnp
from jax import lax
from jax.experimental import pallas as pl
from jax.experimental.pallas import tpu as pltpu
from jax.experimental.pallas import tpu_sc as plsc

N_NODES = 100000
D = 128
NUM_SEGMENTS = 1024
EPS = 1e-5

NC = 2
NS = 16
NW = NC * NS
R = 400
NCHUNK = N_NODES // R
SCW = 80
NSC = R // SCW
CNTW = 16
NB = 2


def _sc_partial_sums(x, batch3, ones_hbm, z_sums, z_cnt):
    mesh = plsc.VectorSubcoreMesh(core_axis_name="c", subcore_axis_name="s")

    @pl.kernel(
        out_type=[
            jax.ShapeDtypeStruct((NC, NUM_SEGMENTS, D), jnp.float32),
            jax.ShapeDtypeStruct((NC, NUM_SEGMENTS, CNTW), jnp.float32),
        ],
        mesh=mesh,
        scratch_types=[
            pltpu.VMEM((NB, R, D), jnp.float32),
            pltpu.VMEM((NB, NSC, SCW), jnp.int32),
            pltpu.VMEM((SCW, CNTW), jnp.float32),
            pltpu.VMEM_SHARED((NUM_SEGMENTS, D), jnp.float32),
            pltpu.VMEM_SHARED((NUM_SEGMENTS, CNTW), jnp.float32),
            pltpu.SemaphoreType.DMA,
            pltpu.SemaphoreType.DMA,
            pltpu.SemaphoreType.DMA,
        ],
    )
    def k(x_hbm, b_hbm, ones_h, zs_h, zc_h, sums_out, cnts_out,
          xbuf, idxbuf, onesbuf, sums_sh, cnts_sh,
          sem0, sem1, sem_sc):
        cid = lax.axis_index("c")
        sid = lax.axis_index("s")
        wid = sid * NC + cid
        sems = (sem0, sem1)

        @pl.when(sid == 0)
        def _():
            pltpu.sync_copy(zs_h, sums_sh)
            pltpu.sync_copy(zc_h, cnts_sh)

        pltpu.sync_copy(ones_h, onesbuf)
        plsc.subcore_barrier()

        lo = (wid * NCHUNK) // NW
        hi = ((wid + 1) * NCHUNK) // NW

        def start_in(chunk, b):
            pltpu.async_copy(x_hbm.at[pl.ds(chunk * R, R)], xbuf.at[b], sems[b])
            pltpu.async_copy(b_hbm.at[chunk], idxbuf.at[b], sems[b])

        def wait_in(chunk, b):
            pltpu.make_async_copy(x_hbm.at[pl.ds(chunk * R, R)], xbuf.at[b],
                                  sems[b]).wait()
            pltpu.make_async_copy(b_hbm.at[chunk], idxbuf.at[b],
                                  sems[b]).wait()

        def fire_scatters(b):
            xv = xbuf.at[b]
            iv = idxbuf.at[b]
            for j in range(NSC):
                pltpu.async_copy(xv.at[pl.ds(j * SCW, SCW)],
                                 sums_sh.at[iv.at[j]], sem_sc, add=True)
                pltpu.async_copy(onesbuf, cnts_sh.at[iv.at[j]], sem_sc,
                                 add=True)

        def drain_scatters(b):
            xv = xbuf.at[b]
            iv = idxbuf.at[b]
            for j in range(NSC):
                pltpu.make_async_copy(xv.at[pl.ds(j * SCW, SCW)],
                                      sums_sh.at[iv.at[j]], sem_sc).wait()
                pltpu.make_async_copy(onesbuf, cnts_sh.at[iv.at[j]],
                                      sem_sc).wait()

        start_in(lo, 0)

        n_outer = (hi - lo + NB - 1) // NB

        def body(kk, carry):
            i = lo + NB * kk
            for b in range(NB):
                chunk = i + b

                @pl.when(chunk < hi)
                def _():
                    bn = (b + 1) % NB

                    @pl.when(chunk - 1 >= lo)
                    def _():
                        drain_scatters(bn)

                    @pl.when(chunk + 1 < hi)
                    def _():
                        start_in(chunk + 1, bn)

                    wait_in(chunk, b)
                    fire_scatters(b)
            return carry

        lax.fori_loop(0, n_outer, body, 0)

        drain_scatters(0)
        plsc.subcore_barrier()

        @pl.when(sid == 0)
        def _():
            pltpu.sync_copy(sums_sh, sums_out.at[cid])
            pltpu.sync_copy(cnts_sh, cnts_out.at[cid])

    return k(x, batch3, ones_hbm, z_sums, z_cnt)


def _finish(sums_ref, cnts_ref, w_ref, b_ref, o_ref):
    s = sums_ref[0] + sums_ref[1]
    c = cnts_ref[0, :, 0:1] + cnts_ref[1, :, 0:1]
    h = s / jnp.maximum(c, 1.0)
    mu = jnp.mean(h, axis=1, keepdims=True)
    var = jnp.mean((h - mu) ** 2, axis=1, keepdims=True)
    o_ref[...] = (h - mu) * lax.rsqrt(var + EPS) * w_ref[0] + b_ref[0]


def kernel(x, batch, ln_weight, ln_bias):
    batch3 = batch.astype(jnp.int32).reshape(NCHUNK, NSC, SCW)
    ones_hbm = jnp.ones((SCW, CNTW), dtype=jnp.float32)
    z_sums = jnp.zeros((NUM_SEGMENTS, D), dtype=jnp.float32)
    z_cnt = jnp.zeros((NUM_SEGMENTS, CNTW), dtype=jnp.float32)

    sums_p, cnts_p = _sc_partial_sums(x, batch3, ones_hbm, z_sums, z_cnt)

    return pl.pallas_call(
        _finish,
        out_shape=jax.ShapeDtypeStruct((NUM_SEGMENTS, D), jnp.float32),
    )(sums_p, cnts_p, ln_weight.reshape(1, D), ln_bias.reshape(1, D))


# speedup vs baseline: 1.1004x; 1.0831x over previous
import jax
import jax.numpy as jnp
from jax import lax
from jax.experimental import pallas as pl
from jax.experimental.pallas import tpu as pltpu
from jax.experimental.pallas import tpu_sc as plsc

N_NODES = 100000
D = 128
NUM_SEGMENTS = 1024
EPS = 1e-5

NC = 2
NS = 16
NW = NC * NS
R = 400
NCHUNK = N_NODES // R
SCW = 80
NSC = R // SCW
CNTW = 16
NB = 2


def _sc_partial_sums(x, batch3, z_sums, z_cnt):
    mesh = plsc.VectorSubcoreMesh(core_axis_name="c", subcore_axis_name="s")

    @pl.kernel(
        out_type=[
            jax.ShapeDtypeStruct((NC, NUM_SEGMENTS, D), jnp.float32),
            jax.ShapeDtypeStruct((NC, NUM_SEGMENTS, CNTW), jnp.float32),
        ],
        mesh=mesh,
        scratch_types=[
            pltpu.VMEM((NB, R, D), jnp.float32),
            pltpu.VMEM((NB, NSC, SCW), jnp.int32),
            pltpu.VMEM((64, CNTW), jnp.float32),
            pltpu.VMEM((4, 16), jnp.int32),
            pltpu.VMEM_SHARED((NUM_SEGMENTS, D), jnp.float32),
            pltpu.VMEM_SHARED((NUM_SEGMENTS + 16, CNTW), jnp.float32),
            pltpu.SemaphoreType.DMA,
            pltpu.SemaphoreType.DMA,
            pltpu.SemaphoreType.DMA,
        ],
    )
    def k(x_hbm, b_hbm, zs_h, zc_h, sums_out, cnts_out,
          xbuf, idxbuf, cntbuf, fbidx, sums_sh, cnts_sh,
          sem0, sem1, sem_sc):
        cid = lax.axis_index("c")
        sid = lax.axis_index("s")
        wid = sid * NC + cid
        sems = (sem0, sem1)
        i32 = jnp.int32
        lanes = lax.iota(i32, 16)
        FMAX = 64
        FG = 4

        @pl.when(sid == 0)
        def _():
            pltpu.sync_copy(zs_h, sums_sh)
            pltpu.sync_copy(zc_h, cnts_sh.at[pl.ds(0, NUM_SEGMENTS)])
            pltpu.sync_copy(zc_h.at[pl.ds(0, 16)],
                            cnts_sh.at[pl.ds(NUM_SEGMENTS, 16)])

        pltpu.sync_copy(zc_h.at[pl.ds(0, 64)], cntbuf)
        for t in range(FG):
            fbidx[t, pl.ds(0, 16)] = NUM_SEGMENTS + lanes
        plsc.subcore_barrier()

        lo = (wid * NCHUNK) // NW
        hi = ((wid + 1) * NCHUNK) // NW

        def start_in(chunk, b):
            pltpu.async_copy(x_hbm.at[pl.ds(chunk * R, R)], xbuf.at[b], sems[b])
            pltpu.async_copy(b_hbm.at[chunk], idxbuf.at[b], sems[b])

        def wait_in(chunk, b):
            pltpu.make_async_copy(x_hbm.at[pl.ds(chunk * R, R)], xbuf.at[b],
                                  sems[b]).wait()
            pltpu.make_async_copy(b_hbm.at[chunk], idxbuf.at[b],
                                  sems[b]).wait()

        def fire_scatters(b):
            xv = xbuf.at[b]
            iv = idxbuf.at[b]
            for j in range(NSC):
                pltpu.async_copy(xv.at[pl.ds(j * SCW, SCW)],
                                 sums_sh.at[iv.at[j]], sem_sc, add=True)

        def drain_scatters(b):
            xv = xbuf.at[b]
            iv = idxbuf.at[b]
            for j in range(NSC):
                pltpu.make_async_copy(xv.at[pl.ds(j * SCW, SCW)],
                                      sums_sh.at[iv.at[j]], sem_sc).wait()

        start_in(lo, 0)

        def flush_cnt(fc, run_id, run_len):
            cntbuf[fc, pl.ds(0, 16)] = jnp.where(lanes == 0, run_len, 0.0)
            fg = fc // 16
            sel = lanes == (fc % 16)
            fbidx[fg, pl.ds(0, 16)] = jnp.where(sel, run_id,
                                                fbidx[fg, pl.ds(0, 16)])
            return fc + 1

        def scatter_cnt(fc):
            ngrp = (fc + 15) // 16
            for t in range(FG):
                @pl.when(t < ngrp)
                def _():
                    pltpu.async_copy(cntbuf.at[pl.ds(t * 16, 16)],
                                     cnts_sh.at[fbidx.at[t]], sem_sc,
                                     add=True).wait()

        def reset_cnt():
            pltpu.sync_copy(zc_h.at[pl.ds(0, 64)], cntbuf)
            for t in range(FG):
                fbidx[t, pl.ds(0, 16)] = NUM_SEGMENTS + lanes

        def count_runs(b, carry):
            def group_body(g, carry2):
                run_id, run_len, fc = carry2
                gj = g // (SCW // 16)
                gk = g % (SCW // 16)
                idv = idxbuf[b, gj, pl.ds(gk * 16, 16)]
                uniform = jnp.logical_and(idv[0] == run_id,
                                          idv[15] == run_id)

                def fast():
                    return run_id, run_len + 16.0, fc

                def slow():
                    rid, rlen, f = run_id, run_len, fc
                    for r in range(16):
                        idr = idv[r]
                        is_new = idr != rid

                        def fl(f=f, rid=rid, rlen=rlen):
                            return flush_cnt(f, rid, rlen)

                        f = lax.cond(jnp.logical_and(is_new, rlen > 0.0),
                                     fl, lambda f=f: f)
                        rlen = jnp.where(is_new, 1.0, rlen + 1.0)
                        rid = idr
                    return rid, rlen, f

                rid3, rlen3, fc3 = lax.cond(uniform, fast, slow)

                def mid(fc3=fc3):
                    scatter_cnt(fc3)
                    reset_cnt()
                    return jnp.int32(0)

                fc3 = lax.cond(fc3 >= FMAX - 16, mid, lambda fc3=fc3: fc3)
                return rid3, rlen3, fc3

            return lax.fori_loop(0, R // 16, group_body, carry)

        n_outer = (hi - lo + NB - 1) // NB

        def body(kk, carry):
            i = lo + NB * kk
            for b in range(NB):
                chunk = i + b

                @pl.when(chunk < hi)
                def _():
                    bn = (b + 1) % NB

                    @pl.when(chunk - 1 >= lo)
                    def _():
                        drain_scatters(bn)

                    @pl.when(chunk + 1 < hi)
                    def _():
                        start_in(chunk + 1, bn)

                    wait_in(chunk, b)
                    fire_scatters(b)

                run_id, run_len, fc = carry
                carry = lax.cond(chunk < hi,
                                 lambda: count_runs(b, (run_id, run_len, fc)),
                                 lambda: (run_id, run_len, fc))
            return carry

        init = (jnp.int32(-1), jnp.float32(0.0), jnp.int32(0))
        run_id, run_len, fc = lax.fori_loop(0, n_outer, body, init)
        fc = lax.cond(run_len > 0.0,
                      lambda: flush_cnt(fc, run_id, run_len),
                      lambda: fc)
        scatter_cnt(fc)

        drain_scatters(0)
        plsc.subcore_barrier()

        @pl.when(sid == 0)
        def _():
            pltpu.sync_copy(sums_sh, sums_out.at[cid])
            pltpu.sync_copy(cnts_sh.at[pl.ds(0, NUM_SEGMENTS)],
                            cnts_out.at[cid])

    return k(x, batch3, z_sums, z_cnt)


def _finish(sums_ref, cnts_ref, w_ref, b_ref, o_ref):
    s = sums_ref[0] + sums_ref[1]
    c = cnts_ref[0, :, 0:1] + cnts_ref[1, :, 0:1]
    h = s / jnp.maximum(c, 1.0)
    mu = jnp.mean(h, axis=1, keepdims=True)
    var = jnp.mean((h - mu) ** 2, axis=1, keepdims=True)
    o_ref[...] = (h - mu) * lax.rsqrt(var + EPS) * w_ref[0] + b_ref[0]


def kernel(x, batch, ln_weight, ln_bias):
    batch3 = batch.astype(jnp.int32).reshape(NCHUNK, NSC, SCW)
    z_sums = jnp.zeros((NUM_SEGMENTS, D), dtype=jnp.float32)
    z_cnt = jnp.zeros((NUM_SEGMENTS, CNTW), dtype=jnp.float32)

    sums_p, cnts_p = _sc_partial_sums(x, batch3, z_sums, z_cnt)

    return pl.pallas_call(
        _finish,
        out_shape=jax.ShapeDtypeStruct((NUM_SEGMENTS, D), jnp.float32),
    )(sums_p, cnts_p, ln_weight.reshape(1, D), ln_bias.reshape(1, D))
